# baseline (speedup 1.0000x reference)
import jax
import jax.numpy as jnp
import numpy as np
from jax import lax
from jax.experimental import pallas as pl
from jax.experimental.pallas import tpu as pltpu

_B, _C, _H, _W = 32, 3, 32, 32
_F = 160
_F2 = 2 * _F
_NBLK = 5
_NMIX = 10
_P = 100
_BB = 4
_NB = _B // _BB
_N = _BB * _H * _W
_NT = _NB * _N
_ROW = _BB * _W
_CHUNKS = 2

_f32 = jnp.float32
_bf16 = jnp.bfloat16


def _wmasks():
    wpos = lax.broadcasted_iota(jnp.int32, (1, _N), 1) % _W
    wm_l = (wpos >= 1).astype(_f32)
    wm_r = (wpos <= _W - 2).astype(_f32)
    return wm_l, wm_r


def _variants(z, dws, wm_l, wm_r):
    vm = {}
    for dw in dws:
        if dw == 0:
            vm[0] = z.astype(_bf16)
        elif dw == -1:
            vm[-1] = (pltpu.roll(z, 1, axis=1) * wm_l).astype(_bf16)
        else:
            vm[1] = (pltpu.roll(z, _N - 1, axis=1) * wm_r).astype(_bf16)
    return vm


def _shift_h(z):
    return jnp.concatenate([jnp.zeros_like(z[:, :_ROW]), z[:, :-_ROW]], axis=1)


def _conv(w2d, vm, dws, two_rows, bias, chunks=_CHUNKS, extra=None):
    cw = _N // chunks
    outs = []
    for c in range(chunks):
        s, e = c * cw, (c + 1) * cw
        parts = []
        for dw in dws:
            v = vm[dw]
            if two_rows:
                if s == 0:
                    top = jnp.concatenate(
                        [jnp.zeros_like(v[:, :_ROW]), v[:, : e - _ROW]], axis=1)
                else:
                    top = v[:, s - _ROW: e - _ROW]
                parts.append(top)
            parts.append(v[:, s:e])
        if extra is not None:
            parts.append(extra[:, s:e])
        op = parts[0] if len(parts) == 1 else jnp.concatenate(parts, axis=0)
        outs.append(jnp.dot(w2d, op, preferred_element_type=_f32))
    y = outs[0] if chunks == 1 else jnp.concatenate(outs, axis=1)
    return y + bias


def _celu(z):
    m = z > 0
    em1 = jnp.exp(-jnp.abs(z)) - 1.0
    return jnp.concatenate(
        [jnp.where(m, z, em1), jnp.where(m, em1, -z)], axis=0)


def _elu(z):
    return jnp.where(z > 0, z, jnp.exp(z) - 1.0)


def _init_body(xq_ref, wu_ref, wd_ref, wr_ref, bu_ref, bd_ref, br_ref,
               u0_ref, ul0_ref):
    wm_l, wm_r = _wmasks()
    xp = xq_ref[...]
    vm = _variants(xp, (-1, 0, 1), wm_l, wm_r)
    uc = _conv(wu_ref[...], vm, (-1, 0, 1), True, bu_ref[...], chunks=1)
    dc = _conv(wd_ref[...], vm, (-1, 0, 1), False, bd_ref[...], chunks=1)
    rc = _conv(wr_ref[...], vm, (0,), True, br_ref[...], chunks=1)
    u0 = _shift_h(uc)
    ul0 = _shift_h(dc) + pltpu.roll(rc, 1, axis=1) * wm_l
    u0_ref[...] = u0.astype(_bf16)
    ul0_ref[...] = ul0.astype(_bf16)


def _main_body(u0_ref, ul0_ref, uw1_ref, ub1_ref, uw2_ref, ub2_ref,
               lw1_ref, lb1_ref, lw2_ref, lb2_ref,
               ulf_ref, u_s, ul_s):
    j = pl.program_id(1)
    wm_l, wm_r = _wmasks()

    @pl.when(j == 0)
    def _():
        u_s[...] = u0_ref[...].astype(_f32)
        ul_s[...] = ul0_ref[...].astype(_f32)

    u = u_s[...]
    ul = ul_s[...]

    xe = _celu(u)
    vm = _variants(xe, (-1, 0, 1), wm_l, wm_r)
    c1 = _conv(uw1_ref[0], vm, (-1, 0, 1), True, ub1_ref[0])
    xe2 = _celu(c1)
    vm2 = _variants(xe2, (-1, 0, 1), wm_l, wm_r)
    c2 = _conv(uw2_ref[0], vm2, (-1, 0, 1), True, ub2_ref[0])
    u_new = u + c2[:_F] * jax.nn.sigmoid(c2[_F:])
    u_s[...] = u_new

    ye = _celu(ul)
    vm3 = _variants(ye, (-1, 0), wm_l, wm_r)
    a = _celu(u_new).astype(_bf16)
    d1 = _conv(lw1_ref[0], vm3, (-1, 0), True, lb1_ref[0], extra=a)
    ye2 = _celu(d1)
    vm4 = _variants(ye2, (-1, 0), wm_l, wm_r)
    d2 = _conv(lw2_ref[0], vm4, (-1, 0), True, lb2_ref[0])
    ul_new = ul + d2[:_F] * jax.nn.sigmoid(d2[_F:])
    ul_s[...] = ul_new

    @pl.when(j == _NBLK - 1)
    def _():
        ulf_ref[...] = ul_new.astype(_bf16)


def _logistic_logp(xc, means, ls):
    inv = jnp.exp(-ls)
    centered = xc - means
    plus_in = inv * (centered + 1.0 / 255.0)
    min_in = inv * (centered - 1.0 / 255.0)
    cdf_delta = jax.nn.sigmoid(plus_in) - jax.nn.sigmoid(min_in)
    log_cdf_plus = plus_in - jax.nn.softplus(plus_in)
    log_om_cdf = -jax.nn.softplus(min_in)
    mid_in = inv * centered
    log_pdf_mid = mid_in - ls - 2.0 * jax.nn.softplus(mid_in)
    return jnp.where(
        xc < -0.999, log_cdf_plus,
        jnp.where(xc > 0.999, log_om_cdf,
                  jnp.where(cdf_delta > 1e-5,
                            jnp.log(jnp.maximum(cdf_delta, 1e-12)),
                            log_pdf_mid - np.float32(np.log(127.5)))))


def _epi_body(ulf_ref, x3_ref, ow_ref, ob_ref, out_ref):
    ul = ulf_ref[...].astype(_f32)
    z = _elu(ul).astype(_bf16)
    l = jnp.dot(ow_ref[...], z, preferred_element_type=_f32) + ob_ref[...]
    x = x3_ref[...]
    x0, x1 = x[0:1], x[1:2]
    xs = (x[0:1], x[1:2], x[2:3])
    m1 = l[10:20]
    m2 = l[40:50] + jnp.tanh(l[30:40]) * x0
    m3 = l[70:80] + jnp.tanh(l[60:70]) * x0 + jnp.tanh(l[90:100]) * x1
    lss = (jnp.maximum(l[20:30], -7.0),
           jnp.maximum(l[50:60], -7.0),
           jnp.maximum(l[80:90], -7.0))
    acc = (_logistic_logp(xs[0], m1, lss[0])
           + _logistic_logp(xs[1], m2, lss[1])
           + _logistic_logp(xs[2], m3, lss[2]))
    lg = l[0:10]
    lgm = lg - jnp.max(lg, axis=0, keepdims=True)
    lsm = lgm - jnp.log(jnp.sum(jnp.exp(lgm), axis=0, keepdims=True))
    tot = acc + lsm
    tm = jnp.max(tot, axis=0, keepdims=True)
    lse = tm + jnp.log(jnp.sum(jnp.exp(tot - tm), axis=0, keepdims=True))
    s = jnp.sum(lse, axis=1, keepdims=True)
    out_ref[...] = jnp.broadcast_to(s[None], (1, 1, 128))


def _pix(a):
    c = a.shape[1]
    return (a.reshape(_NB, _BB, c, _H, _W)
             .transpose(2, 0, 3, 1, 4)
             .reshape(c, _NT))


def _impl(samples, w_u_init, b_u_init, w_ul_d, b_ul_d, w_ul_dr, b_ul_dr,
          u_c1_w, u_c1_b, u_c2_w, u_c2_b,
          ul_c1_w, ul_c1_b, ul_nin_w, ul_nin_b, ul_c2_w, ul_c2_b,
          nin_out_w, nin_out_b, interpret=False):
    x = samples * 2.0 - 1.0
    xp = jnp.concatenate([x, jnp.ones_like(x[:, :1])], axis=1)
    xq = _pix(xp)
    x3 = _pix(x)

    def kstack(w):
        o = w.shape[0]
        return w.transpose(0, 3, 2, 1).reshape(o, -1).astype(_bf16)

    def kstack5(w):
        n, o = w.shape[0], w.shape[1]
        return w.transpose(0, 1, 4, 3, 2).reshape(n, o, -1).astype(_bf16)

    wu0 = kstack(w_u_init)
    wd0 = kstack(w_ul_d)
    wr0 = kstack(w_ul_dr)
    bu0 = b_u_init.reshape(_F, 1)
    bd0 = b_ul_d.reshape(_F, 1)
    br0 = b_ul_dr.reshape(_F, 1)
    uw1 = kstack5(u_c1_w)
    uw2 = kstack5(u_c2_w)
    lw1 = jnp.concatenate(
        [kstack5(ul_c1_w), ul_nin_w.astype(_bf16)], axis=2)
    lw2 = kstack5(ul_c2_w)
    ub1 = u_c1_b.reshape(_NBLK, _F, 1)
    ub2 = u_c2_b.reshape(_NBLK, _F2, 1)
    lb1 = (ul_c1_b + ul_nin_b).reshape(_NBLK, _F, 1)
    lb2 = ul_c2_b.reshape(_NBLK, _F2, 1)
    ow = nin_out_w.astype(_bf16)
    ob = nin_out_b.reshape(_P, 1)

    cp1 = pltpu.CompilerParams(dimension_semantics=("parallel",),
                               vmem_limit_bytes=50 * 1024 * 1024)
    cp2 = pltpu.CompilerParams(dimension_semantics=("parallel", "arbitrary"),
                               vmem_limit_bytes=52 * 1024 * 1024)

    u0, ul0 = pl.pallas_call(
        _init_body,
        grid=(_NB,),
        in_specs=[
            pl.BlockSpec((4, _N), lambda i: (0, i)),
            pl.BlockSpec((_F, 24), lambda i: (0, 0)),
            pl.BlockSpec((_F, 12), lambda i: (0, 0)),
            pl.BlockSpec((_F, 8), lambda i: (0, 0)),
            pl.BlockSpec((_F, 1), lambda i: (0, 0)),
            pl.BlockSpec((_F, 1), lambda i: (0, 0)),
            pl.BlockSpec((_F, 1), lambda i: (0, 0)),
        ],
        out_specs=[pl.BlockSpec((_F, _N), lambda i: (0, i))] * 2,
        out_shape=[jax.ShapeDtypeStruct((_F, _NT), _bf16)] * 2,
        compiler_params=cp1,
        name="pixcnn_init",
        interpret=interpret,
    )(xq, wu0, wd0, wr0, bu0, bd0, br0)

    ulf = pl.pallas_call(
        _main_body,
        grid=(_NB, _NBLK),
        in_specs=[
            pl.BlockSpec((_F, _N), lambda i, j: (0, i)),
            pl.BlockSpec((_F, _N), lambda i, j: (0, i)),
            pl.BlockSpec((1, _F, 6 * _F2), lambda i, j: (j, 0, 0)),
            pl.BlockSpec((1, _F, 1), lambda i, j: (j, 0, 0)),
            pl.BlockSpec((1, _F2, 6 * _F2), lambda i, j: (j, 0, 0)),
            pl.BlockSpec((1, _F2, 1), lambda i, j: (j, 0, 0)),
            pl.BlockSpec((1, _F, 5 * _F2), lambda i, j: (j, 0, 0)),
            pl.BlockSpec((1, _F, 1), lambda i, j: (j, 0, 0)),
            pl.BlockSpec((1, _F2, 4 * _F2), lambda i, j: (j, 0, 0)),
            pl.BlockSpec((1, _F2, 1), lambda i, j: (j, 0, 0)),
        ],
        out_specs=pl.BlockSpec((_F, _N), lambda i, j: (0, i)),
        out_shape=jax.ShapeDtypeStruct((_F, _NT), _bf16),
        scratch_shapes=[pltpu.VMEM((_F, _N), _f32),
                        pltpu.VMEM((_F, _N), _f32)],
        compiler_params=cp2,
        name="pixcnn_resnets",
        interpret=interpret,
    )(u0, ul0, uw1, ub1, uw2, ub2, lw1, lb1, lw2, lb2)

    parts = pl.pallas_call(
        _epi_body,
        grid=(_NB,),
        in_specs=[
            pl.BlockSpec((_F, _N), lambda i: (0, i)),
            pl.BlockSpec((3, _N), lambda i: (0, i)),
            pl.BlockSpec((_P, _F), lambda i: (0, 0)),
            pl.BlockSpec((_P, 1), lambda i: (0, 0)),
        ],
        out_specs=pl.BlockSpec((1, 1, 128), lambda i: (i, 0, 0)),
        out_shape=jax.ShapeDtypeStruct((_NB, 1, 128), _f32),
        compiler_params=cp1,
        name="pixcnn_dmll",
        interpret=interpret,
    )(ulf, x3, ow, ob)

    return jnp.sum(parts[:, 0, 0])


def kernel(samples, w_u_init, b_u_init, w_ul_d, b_ul_d, w_ul_dr, b_ul_dr,
           u_c1_w, u_c1_b, u_c2_w, u_c2_b,
           ul_c1_w, ul_c1_b, ul_nin_w, ul_nin_b, ul_c2_w, ul_c2_b,
           nin_out_w, nin_out_b):
    return _impl(samples, w_u_init, b_u_init, w_ul_d, b_ul_d, w_ul_dr,
                 b_ul_dr, u_c1_w, u_c1_b, u_c2_w, u_c2_b,
                 ul_c1_w, ul_c1_b, ul_nin_w, ul_nin_b, ul_c2_w, ul_c2_b,
                 nin_out_w, nin_out_b)


# speedup vs baseline: 1.0588x; 1.0588x over previous
import jax
import jax.numpy as jnp
import numpy as np
from jax import lax
from jax.experimental import pallas as pl
from jax.experimental.pallas import tpu as pltpu

_B, _C, _H, _W = 32, 3, 32, 32
_F = 160
_F2 = 2 * _F
_NBLK = 5
_NMIX = 10
_P = 100
_BB = 4
_NB = _B // _BB
_N = _BB * _H * _W
_NT = _NB * _N
_ROW = _BB * _W
_CHUNKS = 2

_f32 = jnp.float32
_bf16 = jnp.bfloat16


def _wmasks():
    wpos = lax.broadcasted_iota(jnp.int32, (1, _N), 1) % _W
    wm_l = jnp.where(wpos >= 1, -1, 0)
    wm_r = jnp.where(wpos <= _W - 2, -1, 0)
    return wm_l, wm_r


def _variants(z, dws, wm_l, wm_r):
    zb = z.astype(_bf16)
    zi = pltpu.bitcast(zb, jnp.int32)
    vm = {}
    for dw in dws:
        if dw == 0:
            vm[0] = zb
        elif dw == -1:
            vm[-1] = pltpu.bitcast(pltpu.roll(zi, 1, axis=1) & wm_l, _bf16)
        else:
            vm[1] = pltpu.bitcast(pltpu.roll(zi, _N - 1, axis=1) & wm_r, _bf16)
    return vm


def _shift_h(z):
    return jnp.concatenate([jnp.zeros_like(z[:, :_ROW]), z[:, :-_ROW]], axis=1)


def _conv(w2d, vm, dws, two_rows, bias, chunks=_CHUNKS, extra=None):
    cw = _N // chunks
    outs = []
    for c in range(chunks):
        s, e = c * cw, (c + 1) * cw
        parts = []
        for dw in dws:
            v = vm[dw]
            if two_rows:
                if s == 0:
                    top = jnp.concatenate(
                        [jnp.zeros_like(v[:, :_ROW]), v[:, : e - _ROW]], axis=1)
                else:
                    top = v[:, s - _ROW: e - _ROW]
                parts.append(top)
            parts.append(v[:, s:e])
        if extra is not None:
            parts.append(extra[:, s:e])
        op = parts[0] if len(parts) == 1 else jnp.concatenate(parts, axis=0)
        outs.append(jnp.dot(w2d, op, preferred_element_type=_f32))
    y = outs[0] if chunks == 1 else jnp.concatenate(outs, axis=1)
    return y + bias


def _celu(z):
    zz = jnp.concatenate([z, -z], axis=0)
    return jnp.where(zz > 0, zz, jnp.exp(zz) - 1.0)


def _elu(z):
    return jnp.where(z > 0, z, jnp.exp(z) - 1.0)


def _init_body(xq_ref, wu_ref, wd_ref, wr_ref, bu_ref, bd_ref, br_ref,
               u0_ref, ul0_ref):
    wm_l, wm_r = _wmasks()
    xp = xq_ref[...]
    vm = _variants(xp, (-1, 0, 1), wm_l, wm_r)
    uc = _conv(wu_ref[...], vm, (-1, 0, 1), True, bu_ref[...], chunks=1)
    dc = _conv(wd_ref[...], vm, (-1, 0, 1), False, bd_ref[...], chunks=1)
    rc = _conv(wr_ref[...], vm, (0,), True, br_ref[...], chunks=1)
    u0 = _shift_h(uc)
    rs = pltpu.bitcast(
        pltpu.roll(pltpu.bitcast(rc, jnp.int32), 1, axis=1) & wm_l, _f32)
    ul0 = _shift_h(dc) + rs
    u0_ref[...] = u0.astype(_bf16)
    ul0_ref[...] = ul0.astype(_bf16)


def _main_body(u0_ref, ul0_ref, uw1_ref, ub1_ref, uw2_ref, ub2_ref,
               lw1_ref, lb1_ref, lw2_ref, lb2_ref,
               ulf_ref, u_s, ul_s):
    j = pl.program_id(1)
    wm_l, wm_r = _wmasks()

    @pl.when(j == 0)
    def _():
        u_s[...] = u0_ref[...].astype(_f32)
        ul_s[...] = ul0_ref[...].astype(_f32)

    u = u_s[...]
    ul = ul_s[...]

    xe = _celu(u)
    vm = _variants(xe, (-1, 0, 1), wm_l, wm_r)
    c1 = _conv(uw1_ref[0], vm, (-1, 0, 1), True, ub1_ref[0])
    xe2 = _celu(c1)
    vm2 = _variants(xe2, (-1, 0, 1), wm_l, wm_r)
    c2 = _conv(uw2_ref[0], vm2, (-1, 0, 1), True, ub2_ref[0])
    u_new = u + c2[:_F] * jax.nn.sigmoid(c2[_F:])
    u_s[...] = u_new

    ye = _celu(ul)
    vm3 = _variants(ye, (-1, 0), wm_l, wm_r)
    a = _celu(u_new).astype(_bf16)
    d1 = _conv(lw1_ref[0], vm3, (-1, 0), True, lb1_ref[0], extra=a)
    ye2 = _celu(d1)
    vm4 = _variants(ye2, (-1, 0), wm_l, wm_r)
    d2 = _conv(lw2_ref[0], vm4, (-1, 0), True, lb2_ref[0])
    ul_new = ul + d2[:_F] * jax.nn.sigmoid(d2[_F:])
    ul_s[...] = ul_new

    @pl.when(j == _NBLK - 1)
    def _():
        ulf_ref[...] = ul_new.astype(_bf16)


def _logistic_logp(xc, means, ls):
    inv = jnp.exp(-ls)
    centered = xc - means
    plus_in = inv * (centered + 1.0 / 255.0)
    min_in = inv * (centered - 1.0 / 255.0)
    cdf_delta = jax.nn.sigmoid(plus_in) - jax.nn.sigmoid(min_in)
    log_cdf_plus = plus_in - jax.nn.softplus(plus_in)
    log_om_cdf = -jax.nn.softplus(min_in)
    mid_in = inv * centered
    log_pdf_mid = mid_in - ls - 2.0 * jax.nn.softplus(mid_in)
    return jnp.where(
        xc < -0.999, log_cdf_plus,
        jnp.where(xc > 0.999, log_om_cdf,
                  jnp.where(cdf_delta > 1e-5,
                            jnp.log(jnp.maximum(cdf_delta, 1e-12)),
                            log_pdf_mid - np.float32(np.log(127.5)))))


def _epi_body(ulf_ref, x3_ref, ow_ref, ob_ref, out_ref):
    ul = ulf_ref[...].astype(_f32)
    z = _elu(ul).astype(_bf16)
    l = jnp.dot(ow_ref[...], z, preferred_element_type=_f32) + ob_ref[...]
    x = x3_ref[...]
    x0, x1 = x[0:1], x[1:2]
    xs = (x[0:1], x[1:2], x[2:3])
    m1 = l[10:20]
    m2 = l[40:50] + jnp.tanh(l[30:40]) * x0
    m3 = l[70:80] + jnp.tanh(l[60:70]) * x0 + jnp.tanh(l[90:100]) * x1
    lss = (jnp.maximum(l[20:30], -7.0),
           jnp.maximum(l[50:60], -7.0),
           jnp.maximum(l[80:90], -7.0))
    acc = (_logistic_logp(xs[0], m1, lss[0])
           + _logistic_logp(xs[1], m2, lss[1])
           + _logistic_logp(xs[2], m3, lss[2]))
    lg = l[0:10]
    lgm = lg - jnp.max(lg, axis=0, keepdims=True)
    lsm = lgm - jnp.log(jnp.sum(jnp.exp(lgm), axis=0, keepdims=True))
    tot = acc + lsm
    tm = jnp.max(tot, axis=0, keepdims=True)
    lse = tm + jnp.log(jnp.sum(jnp.exp(tot - tm), axis=0, keepdims=True))
    s = jnp.sum(lse, axis=1, keepdims=True)
    out_ref[...] = jnp.broadcast_to(s[None], (1, 1, 128))


def _pix(a):
    c = a.shape[1]
    return (a.reshape(_NB, _BB, c, _H, _W)
             .transpose(2, 0, 3, 1, 4)
             .reshape(c, _NT))


def _impl(samples, w_u_init, b_u_init, w_ul_d, b_ul_d, w_ul_dr, b_ul_dr,
          u_c1_w, u_c1_b, u_c2_w, u_c2_b,
          ul_c1_w, ul_c1_b, ul_nin_w, ul_nin_b, ul_c2_w, ul_c2_b,
          nin_out_w, nin_out_b, interpret=False):
    x = samples * 2.0 - 1.0
    xp = jnp.concatenate([x, jnp.ones_like(x[:, :1])], axis=1)
    xq = _pix(xp)
    x3 = _pix(x)

    def kstack(w):
        o = w.shape[0]
        return w.transpose(0, 3, 2, 1).reshape(o, -1).astype(_bf16)

    def kstack5(w):
        n, o = w.shape[0], w.shape[1]
        return w.transpose(0, 1, 4, 3, 2).reshape(n, o, -1).astype(_bf16)

    wu0 = kstack(w_u_init)
    wd0 = kstack(w_ul_d)
    wr0 = kstack(w_ul_dr)
    bu0 = b_u_init.reshape(_F, 1)
    bd0 = b_ul_d.reshape(_F, 1)
    br0 = b_ul_dr.reshape(_F, 1)
    uw1 = kstack5(u_c1_w)
    uw2 = kstack5(u_c2_w)
    lw1 = jnp.concatenate(
        [kstack5(ul_c1_w), ul_nin_w.astype(_bf16)], axis=2)
    lw2 = kstack5(ul_c2_w)
    ub1 = u_c1_b.reshape(_NBLK, _F, 1)
    ub2 = u_c2_b.reshape(_NBLK, _F2, 1)
    lb1 = (ul_c1_b + ul_nin_b).reshape(_NBLK, _F, 1)
    lb2 = ul_c2_b.reshape(_NBLK, _F2, 1)
    ow = nin_out_w.astype(_bf16)
    ob = nin_out_b.reshape(_P, 1)

    cp1 = pltpu.CompilerParams(dimension_semantics=("parallel",),
                               vmem_limit_bytes=50 * 1024 * 1024)
    cp2 = pltpu.CompilerParams(dimension_semantics=("parallel", "arbitrary"),
                               vmem_limit_bytes=52 * 1024 * 1024)

    u0, ul0 = pl.pallas_call(
        _init_body,
        grid=(_NB,),
        in_specs=[
            pl.BlockSpec((4, _N), lambda i: (0, i)),
            pl.BlockSpec((_F, 24), lambda i: (0, 0)),
            pl.BlockSpec((_F, 12), lambda i: (0, 0)),
            pl.BlockSpec((_F, 8), lambda i: (0, 0)),
            pl.BlockSpec((_F, 1), lambda i: (0, 0)),
            pl.BlockSpec((_F, 1), lambda i: (0, 0)),
            pl.BlockSpec((_F, 1), lambda i: (0, 0)),
        ],
        out_specs=[pl.BlockSpec((_F, _N), lambda i: (0, i))] * 2,
        out_shape=[jax.ShapeDtypeStruct((_F, _NT), _bf16)] * 2,
        compiler_params=cp1,
        name="pixcnn_init",
        interpret=interpret,
    )(xq, wu0, wd0, wr0, bu0, bd0, br0)

    ulf = pl.pallas_call(
        _main_body,
        grid=(_NB, _NBLK),
        in_specs=[
            pl.BlockSpec((_F, _N), lambda i, j: (0, i)),
            pl.BlockSpec((_F, _N), lambda i, j: (0, i)),
            pl.BlockSpec((1, _F, 6 * _F2), lambda i, j: (j, 0, 0)),
            pl.BlockSpec((1, _F, 1), lambda i, j: (j, 0, 0)),
            pl.BlockSpec((1, _F2, 6 * _F2), lambda i, j: (j, 0, 0)),
            pl.BlockSpec((1, _F2, 1), lambda i, j: (j, 0, 0)),
            pl.BlockSpec((1, _F, 5 * _F2), lambda i, j: (j, 0, 0)),
            pl.BlockSpec((1, _F, 1), lambda i, j: (j, 0, 0)),
            pl.BlockSpec((1, _F2, 4 * _F2), lambda i, j: (j, 0, 0)),
            pl.BlockSpec((1, _F2, 1), lambda i, j: (j, 0, 0)),
        ],
        out_specs=pl.BlockSpec((_F, _N), lambda i, j: (0, i)),
        out_shape=jax.ShapeDtypeStruct((_F, _NT), _bf16),
        scratch_shapes=[pltpu.VMEM((_F, _N), _f32),
                        pltpu.VMEM((_F, _N), _f32)],
        compiler_params=cp2,
        name="pixcnn_resnets",
        interpret=interpret,
    )(u0, ul0, uw1, ub1, uw2, ub2, lw1, lb1, lw2, lb2)

    parts = pl.pallas_call(
        _epi_body,
        grid=(_NB,),
        in_specs=[
            pl.BlockSpec((_F, _N), lambda i: (0, i)),
            pl.BlockSpec((3, _N), lambda i: (0, i)),
            pl.BlockSpec((_P, _F), lambda i: (0, 0)),
            pl.BlockSpec((_P, 1), lambda i: (0, 0)),
        ],
        out_specs=pl.BlockSpec((1, 1, 128), lambda i: (i, 0, 0)),
        out_shape=jax.ShapeDtypeStruct((_NB, 1, 128), _f32),
        compiler_params=cp1,
        name="pixcnn_dmll",
        interpret=interpret,
    )(ulf, x3, ow, ob)

    return jnp.sum(parts[:, 0, 0])


def kernel(samples, w_u_init, b_u_init, w_ul_d, b_ul_d, w_ul_dr, b_ul_dr,
           u_c1_w, u_c1_b, u_c2_w, u_c2_b,
           ul_c1_w, ul_c1_b, ul_nin_w, ul_nin_b, ul_c2_w, ul_c2_b,
           nin_out_w, nin_out_b):
    return _impl(samples, w_u_init, b_u_init, w_ul_d, b_ul_d, w_ul_dr,
                 b_ul_dr, u_c1_w, u_c1_b, u_c2_w, u_c2_b,
                 ul_c1_w, ul_c1_b, ul_nin_w, ul_nin_b, ul_c2_w, ul_c2_b,
                 nin_out_w, nin_out_b)


# speedup vs baseline: 1.0588x; 1.0000x over previous
import jax
import jax.numpy as jnp
import numpy as np
from jax import lax
from jax.experimental import pallas as pl
from jax.experimental.pallas import tpu as pltpu

_B, _C, _H, _W = 32, 3, 32, 32
_F = 160
_F2 = 2 * _F
_NBLK = 5
_NMIX = 10
_P = 100
_BB = 4
_NB = _B // _BB
_N = _BB * _H * _W
_NT = _NB * _N
_ROW = _BB * _W
_CHUNKS = 1

_f32 = jnp.float32
_bf16 = jnp.bfloat16


def _wmasks():
    wpos = lax.broadcasted_iota(jnp.int32, (1, _N), 1) % _W
    wm_l = jnp.where(wpos >= 1, -1, 0)
    wm_r = jnp.where(wpos <= _W - 2, -1, 0)
    return wm_l, wm_r


def _variants(z, dws, wm_l, wm_r):
    zb = z.astype(_bf16)
    zi = pltpu.bitcast(zb, jnp.int32)
    vm = {}
    for dw in dws:
        if dw == 0:
            vm[0] = zb
        elif dw == -1:
            vm[-1] = pltpu.bitcast(pltpu.roll(zi, 1, axis=1) & wm_l, _bf16)
        else:
            vm[1] = pltpu.bitcast(pltpu.roll(zi, _N - 1, axis=1) & wm_r, _bf16)
    return vm


def _shift_h(z):
    return jnp.concatenate([jnp.zeros_like(z[:, :_ROW]), z[:, :-_ROW]], axis=1)


def _conv(w2d, vm, dws, two_rows, bias, chunks=_CHUNKS, extra=None):
    cw = _N // chunks
    outs = []
    for c in range(chunks):
        s, e = c * cw, (c + 1) * cw
        parts = []
        for dw in dws:
            v = vm[dw]
            if two_rows:
                if s == 0:
                    top = jnp.concatenate(
                        [jnp.zeros_like(v[:, :_ROW]), v[:, : e - _ROW]], axis=1)
                else:
                    top = v[:, s - _ROW: e - _ROW]
                parts.append(top)
            parts.append(v[:, s:e])
        if extra is not None:
            parts.append(extra[:, s:e])
        op = parts[0] if len(parts) == 1 else jnp.concatenate(parts, axis=0)
        outs.append(jnp.dot(w2d, op, preferred_element_type=_f32))
    y = outs[0] if chunks == 1 else jnp.concatenate(outs, axis=1)
    return y + bias


def _celu(z):
    zz = jnp.concatenate([z, -z], axis=0)
    return jnp.where(zz > 0, zz, jnp.exp(zz) - 1.0)


def _elu(z):
    return jnp.where(z > 0, z, jnp.exp(z) - 1.0)


def _init_body(xq_ref, wu_ref, wd_ref, wr_ref, bu_ref, bd_ref, br_ref,
               u0_ref, ul0_ref):
    wm_l, wm_r = _wmasks()
    xp = xq_ref[...]
    vm = _variants(xp, (-1, 0, 1), wm_l, wm_r)
    uc = _conv(wu_ref[...], vm, (-1, 0, 1), True, bu_ref[...], chunks=1)
    dc = _conv(wd_ref[...], vm, (-1, 0, 1), False, bd_ref[...], chunks=1)
    rc = _conv(wr_ref[...], vm, (0,), True, br_ref[...], chunks=1)
    u0 = _shift_h(uc)
    rs = pltpu.bitcast(
        pltpu.roll(pltpu.bitcast(rc, jnp.int32), 1, axis=1) & wm_l, _f32)
    ul0 = _shift_h(dc) + rs
    u0_ref[...] = u0.astype(_bf16)
    ul0_ref[...] = ul0.astype(_bf16)


def _main_body(u0_ref, ul0_ref, uw1_ref, ub1_ref, uw2_ref, ub2_ref,
               lw1_ref, lb1_ref, lw2_ref, lb2_ref,
               ulf_ref, u_s, ul_s):
    j = pl.program_id(1)
    wm_l, wm_r = _wmasks()

    @pl.when(j == 0)
    def _():
        u_s[...] = u0_ref[...].astype(_f32)
        ul_s[...] = ul0_ref[...].astype(_f32)

    u = u_s[...]
    ul = ul_s[...]

    xe = _celu(u)
    vm = _variants(xe, (-1, 0, 1), wm_l, wm_r)
    c1 = _conv(uw1_ref[0], vm, (-1, 0, 1), True, ub1_ref[0])
    xe2 = _celu(c1)
    vm2 = _variants(xe2, (-1, 0, 1), wm_l, wm_r)
    c2 = _conv(uw2_ref[0], vm2, (-1, 0, 1), True, ub2_ref[0])
    u_new = u + c2[:_F] * jax.nn.sigmoid(c2[_F:])
    u_s[...] = u_new

    ye = _celu(ul)
    vm3 = _variants(ye, (-1, 0), wm_l, wm_r)
    a = _celu(u_new).astype(_bf16)
    d1 = _conv(lw1_ref[0], vm3, (-1, 0), True, lb1_ref[0], extra=a)
    ye2 = _celu(d1)
    vm4 = _variants(ye2, (-1, 0), wm_l, wm_r)
    d2 = _conv(lw2_ref[0], vm4, (-1, 0), True, lb2_ref[0])
    ul_new = ul + d2[:_F] * jax.nn.sigmoid(d2[_F:])
    ul_s[...] = ul_new

    @pl.when(j == _NBLK - 1)
    def _():
        ulf_ref[...] = ul_new.astype(_bf16)


def _logistic_logp(xc, means, ls):
    inv = jnp.exp(-ls)
    centered = xc - means
    plus_in = inv * (centered + 1.0 / 255.0)
    min_in = inv * (centered - 1.0 / 255.0)
    cdf_delta = jax.nn.sigmoid(plus_in) - jax.nn.sigmoid(min_in)
    log_cdf_plus = plus_in - jax.nn.softplus(plus_in)
    log_om_cdf = -jax.nn.softplus(min_in)
    mid_in = inv * centered
    log_pdf_mid = mid_in - ls - 2.0 * jax.nn.softplus(mid_in)
    return jnp.where(
        xc < -0.999, log_cdf_plus,
        jnp.where(xc > 0.999, log_om_cdf,
                  jnp.where(cdf_delta > 1e-5,
                            jnp.log(jnp.maximum(cdf_delta, 1e-12)),
                            log_pdf_mid - np.float32(np.log(127.5)))))


def _epi_body(ulf_ref, x3_ref, ow_ref, ob_ref, out_ref):
    ul = ulf_ref[...].astype(_f32)
    z = _elu(ul).astype(_bf16)
    l = jnp.dot(ow_ref[...], z, preferred_element_type=_f32) + ob_ref[...]
    x = x3_ref[...]
    x0, x1 = x[0:1], x[1:2]
    xs = (x[0:1], x[1:2], x[2:3])
    m1 = l[10:20]
    m2 = l[40:50] + jnp.tanh(l[30:40]) * x0
    m3 = l[70:80] + jnp.tanh(l[60:70]) * x0 + jnp.tanh(l[90:100]) * x1
    lss = (jnp.maximum(l[20:30], -7.0),
           jnp.maximum(l[50:60], -7.0),
           jnp.maximum(l[80:90], -7.0))
    acc = (_logistic_logp(xs[0], m1, lss[0])
           + _logistic_logp(xs[1], m2, lss[1])
           + _logistic_logp(xs[2], m3, lss[2]))
    lg = l[0:10]
    lgm = lg - jnp.max(lg, axis=0, keepdims=True)
    lsm = lgm - jnp.log(jnp.sum(jnp.exp(lgm), axis=0, keepdims=True))
    tot = acc + lsm
    tm = jnp.max(tot, axis=0, keepdims=True)
    lse = tm + jnp.log(jnp.sum(jnp.exp(tot - tm), axis=0, keepdims=True))
    s = jnp.sum(lse, axis=1, keepdims=True)
    out_ref[...] = jnp.broadcast_to(s[None], (1, 1, 128))


def _pix(a):
    c = a.shape[1]
    return (a.reshape(_NB, _BB, c, _H, _W)
             .transpose(2, 0, 3, 1, 4)
             .reshape(c, _NT))


def _impl(samples, w_u_init, b_u_init, w_ul_d, b_ul_d, w_ul_dr, b_ul_dr,
          u_c1_w, u_c1_b, u_c2_w, u_c2_b,
          ul_c1_w, ul_c1_b, ul_nin_w, ul_nin_b, ul_c2_w, ul_c2_b,
          nin_out_w, nin_out_b, interpret=False):
    x = samples * 2.0 - 1.0
    xp = jnp.concatenate([x, jnp.ones_like(x[:, :1])], axis=1)
    xq = _pix(xp)
    x3 = _pix(x)

    def kstack(w):
        o = w.shape[0]
        return w.transpose(0, 3, 2, 1).reshape(o, -1).astype(_bf16)

    def kstack5(w):
        n, o = w.shape[0], w.shape[1]
        return w.transpose(0, 1, 4, 3, 2).reshape(n, o, -1).astype(_bf16)

    wu0 = kstack(w_u_init)
    wd0 = kstack(w_ul_d)
    wr0 = kstack(w_ul_dr)
    bu0 = b_u_init.reshape(_F, 1)
    bd0 = b_ul_d.reshape(_F, 1)
    br0 = b_ul_dr.reshape(_F, 1)
    uw1 = kstack5(u_c1_w)
    uw2 = kstack5(u_c2_w)
    lw1 = jnp.concatenate(
        [kstack5(ul_c1_w), ul_nin_w.astype(_bf16)], axis=2)
    lw2 = kstack5(ul_c2_w)
    ub1 = u_c1_b.reshape(_NBLK, _F, 1)
    ub2 = u_c2_b.reshape(_NBLK, _F2, 1)
    lb1 = (ul_c1_b + ul_nin_b).reshape(_NBLK, _F, 1)
    lb2 = ul_c2_b.reshape(_NBLK, _F2, 1)
    ow = nin_out_w.astype(_bf16)
    ob = nin_out_b.reshape(_P, 1)

    cp1 = pltpu.CompilerParams(dimension_semantics=("parallel",),
                               vmem_limit_bytes=50 * 1024 * 1024)
    cp2 = pltpu.CompilerParams(dimension_semantics=("parallel", "arbitrary"),
                               vmem_limit_bytes=52 * 1024 * 1024)

    u0, ul0 = pl.pallas_call(
        _init_body,
        grid=(_NB,),
        in_specs=[
            pl.BlockSpec((4, _N), lambda i: (0, i)),
            pl.BlockSpec((_F, 24), lambda i: (0, 0)),
            pl.BlockSpec((_F, 12), lambda i: (0, 0)),
            pl.BlockSpec((_F, 8), lambda i: (0, 0)),
            pl.BlockSpec((_F, 1), lambda i: (0, 0)),
            pl.BlockSpec((_F, 1), lambda i: (0, 0)),
            pl.BlockSpec((_F, 1), lambda i: (0, 0)),
        ],
        out_specs=[pl.BlockSpec((_F, _N), lambda i: (0, i))] * 2,
        out_shape=[jax.ShapeDtypeStruct((_F, _NT), _bf16)] * 2,
        compiler_params=cp1,
        name="pixcnn_init",
        interpret=interpret,
    )(xq, wu0, wd0, wr0, bu0, bd0, br0)

    ulf = pl.pallas_call(
        _main_body,
        grid=(_NB, _NBLK),
        in_specs=[
            pl.BlockSpec((_F, _N), lambda i, j: (0, i)),
            pl.BlockSpec((_F, _N), lambda i, j: (0, i)),
            pl.BlockSpec((1, _F, 6 * _F2), lambda i, j: (j, 0, 0)),
            pl.BlockSpec((1, _F, 1), lambda i, j: (j, 0, 0)),
            pl.BlockSpec((1, _F2, 6 * _F2), lambda i, j: (j, 0, 0)),
            pl.BlockSpec((1, _F2, 1), lambda i, j: (j, 0, 0)),
            pl.BlockSpec((1, _F, 5 * _F2), lambda i, j: (j, 0, 0)),
            pl.BlockSpec((1, _F, 1), lambda i, j: (j, 0, 0)),
            pl.BlockSpec((1, _F2, 4 * _F2), lambda i, j: (j, 0, 0)),
            pl.BlockSpec((1, _F2, 1), lambda i, j: (j, 0, 0)),
        ],
        out_specs=pl.BlockSpec((_F, _N), lambda i, j: (0, i)),
        out_shape=jax.ShapeDtypeStruct((_F, _NT), _bf16),
        scratch_shapes=[pltpu.VMEM((_F, _N), _f32),
                        pltpu.VMEM((_F, _N), _f32)],
        compiler_params=cp2,
        name="pixcnn_resnets",
        interpret=interpret,
    )(u0, ul0, uw1, ub1, uw2, ub2, lw1, lb1, lw2, lb2)

    parts = pl.pallas_call(
        _epi_body,
        grid=(_NB,),
        in_specs=[
            pl.BlockSpec((_F, _N), lambda i: (0, i)),
            pl.BlockSpec((3, _N), lambda i: (0, i)),
            pl.BlockSpec((_P, _F), lambda i: (0, 0)),
            pl.BlockSpec((_P, 1), lambda i: (0, 0)),
        ],
        out_specs=pl.BlockSpec((1, 1, 128), lambda i: (i, 0, 0)),
        out_shape=jax.ShapeDtypeStruct((_NB, 1, 128), _f32),
        compiler_params=cp1,
        name="pixcnn_dmll",
        interpret=interpret,
    )(ulf, x3, ow, ob)

    return jnp.sum(parts[:, 0, 0])


def kernel(samples, w_u_init, b_u_init, w_ul_d, b_ul_d, w_ul_dr, b_ul_dr,
           u_c1_w, u_c1_b, u_c2_w, u_c2_b,
           ul_c1_w, ul_c1_b, ul_nin_w, ul_nin_b, ul_c2_w, ul_c2_b,
           nin_out_w, nin_out_b):
    return _impl(samples, w_u_init, b_u_init, w_ul_d, b_ul_d, w_ul_dr,
                 b_ul_dr, u_c1_w, u_c1_b, u_c2_w, u_c2_b,
                 ul_c1_w, ul_c1_b, ul_nin_w, ul_nin_b, ul_c2_w, ul_c2_b,
                 nin_out_w, nin_out_b)


# speedup vs baseline: 1.0637x; 1.0047x over previous
import jax
import jax.numpy as jnp
import numpy as np
from jax import lax
from jax.experimental import pallas as pl
from jax.experimental.pallas import tpu as pltpu

_B, _C, _H, _W = 32, 3, 32, 32
_F = 160
_F2 = 2 * _F
_NBLK = 5
_NMIX = 10
_P = 100
_BB = 8
_NB = _B // _BB
_N = _BB * _H * _W
_NT = _NB * _N
_ROW = _BB * _W
_CHUNKS = 1

_f32 = jnp.float32
_bf16 = jnp.bfloat16


def _wmasks():
    wpos = lax.broadcasted_iota(jnp.int32, (1, _N), 1) % _W
    wm_l = jnp.where(wpos >= 1, -1, 0)
    wm_r = jnp.where(wpos <= _W - 2, -1, 0)
    return wm_l, wm_r


def _variants(z, dws, wm_l, wm_r):
    zb = z.astype(_bf16)
    zi = pltpu.bitcast(zb, jnp.int32)
    vm = {}
    for dw in dws:
        if dw == 0:
            vm[0] = zb
        elif dw == -1:
            vm[-1] = pltpu.bitcast(pltpu.roll(zi, 1, axis=1) & wm_l, _bf16)
        else:
            vm[1] = pltpu.bitcast(pltpu.roll(zi, _N - 1, axis=1) & wm_r, _bf16)
    return vm


def _shift_h(z):
    return jnp.concatenate([jnp.zeros_like(z[:, :_ROW]), z[:, :-_ROW]], axis=1)


def _conv(w2d, vm, dws, two_rows, bias, chunks=_CHUNKS, extra=None):
    cw = _N // chunks
    outs = []
    for c in range(chunks):
        s, e = c * cw, (c + 1) * cw
        parts = []
        for dw in dws:
            v = vm[dw]
            if two_rows:
                if s == 0:
                    top = jnp.concatenate(
                        [jnp.zeros_like(v[:, :_ROW]), v[:, : e - _ROW]], axis=1)
                else:
                    top = v[:, s - _ROW: e - _ROW]
                parts.append(top)
            parts.append(v[:, s:e])
        if extra is not None:
            parts.append(extra[:, s:e])
        op = parts[0] if len(parts) == 1 else jnp.concatenate(parts, axis=0)
        outs.append(jnp.dot(w2d, op, preferred_element_type=_f32))
    y = outs[0] if chunks == 1 else jnp.concatenate(outs, axis=1)
    return y + bias


def _celu(z):
    zz = jnp.concatenate([z, -z], axis=0)
    return jnp.where(zz > 0, zz, jnp.exp(zz) - 1.0)


def _elu(z):
    return jnp.where(z > 0, z, jnp.exp(z) - 1.0)


def _init_body(xq_ref, wu_ref, wd_ref, wr_ref, bu_ref, bd_ref, br_ref,
               u0_ref, ul0_ref):
    wm_l, wm_r = _wmasks()
    xp = xq_ref[...]
    vm = _variants(xp, (-1, 0, 1), wm_l, wm_r)
    uc = _conv(wu_ref[...], vm, (-1, 0, 1), True, bu_ref[...], chunks=1)
    dc = _conv(wd_ref[...], vm, (-1, 0, 1), False, bd_ref[...], chunks=1)
    rc = _conv(wr_ref[...], vm, (0,), True, br_ref[...], chunks=1)
    u0 = _shift_h(uc)
    rs = pltpu.bitcast(
        pltpu.roll(pltpu.bitcast(rc, jnp.int32), 1, axis=1) & wm_l, _f32)
    ul0 = _shift_h(dc) + rs
    u0_ref[...] = u0.astype(_bf16)
    ul0_ref[...] = ul0.astype(_bf16)


def _main_body(u0_ref, ul0_ref, uw1_ref, ub1_ref, uw2_ref, ub2_ref,
               lw1_ref, lb1_ref, lw2_ref, lb2_ref,
               ulf_ref, u_s, ul_s):
    j = pl.program_id(1)
    wm_l, wm_r = _wmasks()

    @pl.when(j == 0)
    def _():
        u_s[...] = u0_ref[...].astype(_f32)
        ul_s[...] = ul0_ref[...].astype(_f32)

    u = u_s[...]
    ul = ul_s[...]

    xe = _celu(u)
    vm = _variants(xe, (-1, 0, 1), wm_l, wm_r)
    c1 = _conv(uw1_ref[0], vm, (-1, 0, 1), True, ub1_ref[0])
    xe2 = _celu(c1)
    vm2 = _variants(xe2, (-1, 0, 1), wm_l, wm_r)
    c2 = _conv(uw2_ref[0], vm2, (-1, 0, 1), True, ub2_ref[0])
    u_new = u + c2[:_F] * jax.nn.sigmoid(c2[_F:])
    u_s[...] = u_new

    ye = _celu(ul)
    vm3 = _variants(ye, (-1, 0), wm_l, wm_r)
    a = _celu(u_new).astype(_bf16)
    d1 = _conv(lw1_ref[0], vm3, (-1, 0), True, lb1_ref[0], extra=a)
    ye2 = _celu(d1)
    vm4 = _variants(ye2, (-1, 0), wm_l, wm_r)
    d2 = _conv(lw2_ref[0], vm4, (-1, 0), True, lb2_ref[0])
    ul_new = ul + d2[:_F] * jax.nn.sigmoid(d2[_F:])
    ul_s[...] = ul_new

    @pl.when(j == _NBLK - 1)
    def _():
        ulf_ref[...] = ul_new.astype(_bf16)


def _logistic_logp(xc, means, ls):
    inv = jnp.exp(-ls)
    centered = xc - means
    plus_in = inv * (centered + 1.0 / 255.0)
    min_in = inv * (centered - 1.0 / 255.0)
    cdf_delta = jax.nn.sigmoid(plus_in) - jax.nn.sigmoid(min_in)
    log_cdf_plus = plus_in - jax.nn.softplus(plus_in)
    log_om_cdf = -jax.nn.softplus(min_in)
    mid_in = inv * centered
    log_pdf_mid = mid_in - ls - 2.0 * jax.nn.softplus(mid_in)
    return jnp.where(
        xc < -0.999, log_cdf_plus,
        jnp.where(xc > 0.999, log_om_cdf,
                  jnp.where(cdf_delta > 1e-5,
                            jnp.log(jnp.maximum(cdf_delta, 1e-12)),
                            log_pdf_mid - np.float32(np.log(127.5)))))


def _epi_body(ulf_ref, x3_ref, ow_ref, ob_ref, out_ref):
    ul = ulf_ref[...].astype(_f32)
    z = _elu(ul).astype(_bf16)
    l = jnp.dot(ow_ref[...], z, preferred_element_type=_f32) + ob_ref[...]
    x = x3_ref[...]
    x0, x1 = x[0:1], x[1:2]
    xs = (x[0:1], x[1:2], x[2:3])
    m1 = l[10:20]
    m2 = l[40:50] + jnp.tanh(l[30:40]) * x0
    m3 = l[70:80] + jnp.tanh(l[60:70]) * x0 + jnp.tanh(l[90:100]) * x1
    lss = (jnp.maximum(l[20:30], -7.0),
           jnp.maximum(l[50:60], -7.0),
           jnp.maximum(l[80:90], -7.0))
    acc = (_logistic_logp(xs[0], m1, lss[0])
           + _logistic_logp(xs[1], m2, lss[1])
           + _logistic_logp(xs[2], m3, lss[2]))
    lg = l[0:10]
    lgm = lg - jnp.max(lg, axis=0, keepdims=True)
    lsm = lgm - jnp.log(jnp.sum(jnp.exp(lgm), axis=0, keepdims=True))
    tot = acc + lsm
    tm = jnp.max(tot, axis=0, keepdims=True)
    lse = tm + jnp.log(jnp.sum(jnp.exp(tot - tm), axis=0, keepdims=True))
    s = jnp.sum(lse, axis=1, keepdims=True)
    out_ref[...] = jnp.broadcast_to(s[None], (1, 1, 128))


def _pix(a):
    c = a.shape[1]
    return (a.reshape(_NB, _BB, c, _H, _W)
             .transpose(2, 0, 3, 1, 4)
             .reshape(c, _NT))


def _impl(samples, w_u_init, b_u_init, w_ul_d, b_ul_d, w_ul_dr, b_ul_dr,
          u_c1_w, u_c1_b, u_c2_w, u_c2_b,
          ul_c1_w, ul_c1_b, ul_nin_w, ul_nin_b, ul_c2_w, ul_c2_b,
          nin_out_w, nin_out_b, interpret=False):
    x = samples * 2.0 - 1.0
    xp = jnp.concatenate([x, jnp.ones_like(x[:, :1])], axis=1)
    xq = _pix(xp)
    x3 = _pix(x)

    def kstack(w):
        o = w.shape[0]
        return w.transpose(0, 3, 2, 1).reshape(o, -1).astype(_bf16)

    def kstack5(w):
        n, o = w.shape[0], w.shape[1]
        return w.transpose(0, 1, 4, 3, 2).reshape(n, o, -1).astype(_bf16)

    wu0 = kstack(w_u_init)
    wd0 = kstack(w_ul_d)
    wr0 = kstack(w_ul_dr)
    bu0 = b_u_init.reshape(_F, 1)
    bd0 = b_ul_d.reshape(_F, 1)
    br0 = b_ul_dr.reshape(_F, 1)
    uw1 = kstack5(u_c1_w)
    uw2 = kstack5(u_c2_w)
    lw1 = jnp.concatenate(
        [kstack5(ul_c1_w), ul_nin_w.astype(_bf16)], axis=2)
    lw2 = kstack5(ul_c2_w)
    ub1 = u_c1_b.reshape(_NBLK, _F, 1)
    ub2 = u_c2_b.reshape(_NBLK, _F2, 1)
    lb1 = (ul_c1_b + ul_nin_b).reshape(_NBLK, _F, 1)
    lb2 = ul_c2_b.reshape(_NBLK, _F2, 1)
    ow = nin_out_w.astype(_bf16)
    ob = nin_out_b.reshape(_P, 1)

    cp1 = pltpu.CompilerParams(dimension_semantics=("parallel",),
                               vmem_limit_bytes=50 * 1024 * 1024)
    cp2 = pltpu.CompilerParams(dimension_semantics=("parallel", "arbitrary"),
                               vmem_limit_bytes=56 * 1024 * 1024)

    u0, ul0 = pl.pallas_call(
        _init_body,
        grid=(_NB,),
        in_specs=[
            pl.BlockSpec((4, _N), lambda i: (0, i)),
            pl.BlockSpec((_F, 24), lambda i: (0, 0)),
            pl.BlockSpec((_F, 12), lambda i: (0, 0)),
            pl.BlockSpec((_F, 8), lambda i: (0, 0)),
            pl.BlockSpec((_F, 1), lambda i: (0, 0)),
            pl.BlockSpec((_F, 1), lambda i: (0, 0)),
            pl.BlockSpec((_F, 1), lambda i: (0, 0)),
        ],
        out_specs=[pl.BlockSpec((_F, _N), lambda i: (0, i))] * 2,
        out_shape=[jax.ShapeDtypeStruct((_F, _NT), _bf16)] * 2,
        compiler_params=cp1,
        name="pixcnn_init",
        interpret=interpret,
    )(xq, wu0, wd0, wr0, bu0, bd0, br0)

    ulf = pl.pallas_call(
        _main_body,
        grid=(_NB, _NBLK),
        in_specs=[
            pl.BlockSpec((_F, _N), lambda i, j: (0, i)),
            pl.BlockSpec((_F, _N), lambda i, j: (0, i)),
            pl.BlockSpec((1, _F, 6 * _F2), lambda i, j: (j, 0, 0)),
            pl.BlockSpec((1, _F, 1), lambda i, j: (j, 0, 0)),
            pl.BlockSpec((1, _F2, 6 * _F2), lambda i, j: (j, 0, 0)),
            pl.BlockSpec((1, _F2, 1), lambda i, j: (j, 0, 0)),
            pl.BlockSpec((1, _F, 5 * _F2), lambda i, j: (j, 0, 0)),
            pl.BlockSpec((1, _F, 1), lambda i, j: (j, 0, 0)),
            pl.BlockSpec((1, _F2, 4 * _F2), lambda i, j: (j, 0, 0)),
            pl.BlockSpec((1, _F2, 1), lambda i, j: (j, 0, 0)),
        ],
        out_specs=pl.BlockSpec((_F, _N), lambda i, j: (0, i)),
        out_shape=jax.ShapeDtypeStruct((_F, _NT), _bf16),
        scratch_shapes=[pltpu.VMEM((_F, _N), _f32),
                        pltpu.VMEM((_F, _N), _f32)],
        compiler_params=cp2,
        name="pixcnn_resnets",
        interpret=interpret,
    )(u0, ul0, uw1, ub1, uw2, ub2, lw1, lb1, lw2, lb2)

    parts = pl.pallas_call(
        _epi_body,
        grid=(_NB,),
        in_specs=[
            pl.BlockSpec((_F, _N), lambda i: (0, i)),
            pl.BlockSpec((3, _N), lambda i: (0, i)),
            pl.BlockSpec((_P, _F), lambda i: (0, 0)),
            pl.BlockSpec((_P, 1), lambda i: (0, 0)),
        ],
        out_specs=pl.BlockSpec((1, 1, 128), lambda i: (i, 0, 0)),
        out_shape=jax.ShapeDtypeStruct((_NB, 1, 128), _f32),
        compiler_params=cp1,
        name="pixcnn_dmll",
        interpret=interpret,
    )(ulf, x3, ow, ob)

    return jnp.sum(parts[:, 0, 0])


def kernel(samples, w_u_init, b_u_init, w_ul_d, b_ul_d, w_ul_dr, b_ul_dr,
           u_c1_w, u_c1_b, u_c2_w, u_c2_b,
           ul_c1_w, ul_c1_b, ul_nin_w, ul_nin_b, ul_c2_w, ul_c2_b,
           nin_out_w, nin_out_b):
    return _impl(samples, w_u_init, b_u_init, w_ul_d, b_ul_d, w_ul_dr,
                 b_ul_dr, u_c1_w, u_c1_b, u_c2_w, u_c2_b,
                 ul_c1_w, ul_c1_b, ul_nin_w, ul_nin_b, ul_c2_w, ul_c2_b,
                 nin_out_w, nin_out_b)


# speedup vs baseline: 1.1429x; 1.0744x over previous
import jax
import jax.numpy as jnp
import numpy as np
from jax import lax
from jax.experimental import pallas as pl
from jax.experimental.pallas import tpu as pltpu

_B, _C, _H, _W = 32, 3, 32, 32
_F = 160
_F2 = 2 * _F
_NBLK = 5
_NMIX = 10
_P = 100
_BB = 8
_NB = _B // _BB
_N = _BB * _H * _W
_NT = _NB * _N
_ROW = _BB * _W
_CHUNKS = 1

_f32 = jnp.float32
_bf16 = jnp.bfloat16
_CDT = jnp.float8_e4m3fn
_WSCALE = 16.0
_INV_WSCALE = 1.0 / _WSCALE


def _wmasks():
    wpos = lax.broadcasted_iota(jnp.int32, (1, _N), 1) % _W
    wm_l = jnp.where(wpos >= 1, -1, 0)
    wm_r = jnp.where(wpos <= _W - 2, -1, 0)
    return wm_l, wm_r


def _variants(z, dws, wm_l, wm_r, dt=_bf16):
    zb = z.astype(dt)
    zi = pltpu.bitcast(zb, jnp.int32)
    vm = {}
    for dw in dws:
        if dw == 0:
            vm[0] = zb
        elif dw == -1:
            vm[-1] = pltpu.bitcast(pltpu.roll(zi, 1, axis=1) & wm_l, dt)
        else:
            vm[1] = pltpu.bitcast(pltpu.roll(zi, _N - 1, axis=1) & wm_r, dt)
    return vm


def _shift_h(z):
    return jnp.concatenate([jnp.zeros_like(z[:, :_ROW]), z[:, :-_ROW]], axis=1)


def _conv(w2d, vm, dws, two_rows, bias, chunks=_CHUNKS, extra=None,
          scale=None):
    cw = _N // chunks
    outs = []
    for c in range(chunks):
        s, e = c * cw, (c + 1) * cw
        parts = []
        for dw in dws:
            v = vm[dw]
            if two_rows:
                if s == 0:
                    top = jnp.concatenate(
                        [jnp.zeros_like(v[:, :_ROW]), v[:, : e - _ROW]], axis=1)
                else:
                    top = v[:, s - _ROW: e - _ROW]
                parts.append(top)
            parts.append(v[:, s:e])
        if extra is not None:
            parts.append(extra[:, s:e])
        op = parts[0] if len(parts) == 1 else jnp.concatenate(parts, axis=0)
        outs.append(jnp.dot(w2d, op, preferred_element_type=_f32))
    y = outs[0] if chunks == 1 else jnp.concatenate(outs, axis=1)
    if scale is not None:
        return y * scale + bias
    return y + bias


def _celu(z):
    zz = jnp.concatenate([z, -z], axis=0)
    return jnp.where(zz > 0, zz, jnp.exp(zz) - 1.0)


def _elu(z):
    return jnp.where(z > 0, z, jnp.exp(z) - 1.0)


def _init_body(xq_ref, wu_ref, wd_ref, wr_ref, bu_ref, bd_ref, br_ref,
               u0_ref, ul0_ref):
    wm_l, wm_r = _wmasks()
    xp = xq_ref[...]
    vm = _variants(xp, (-1, 0, 1), wm_l, wm_r)
    uc = _conv(wu_ref[...], vm, (-1, 0, 1), True, bu_ref[...], chunks=1)
    dc = _conv(wd_ref[...], vm, (-1, 0, 1), False, bd_ref[...], chunks=1)
    rc = _conv(wr_ref[...], vm, (0,), True, br_ref[...], chunks=1)
    u0 = _shift_h(uc)
    rs = pltpu.bitcast(
        pltpu.roll(pltpu.bitcast(rc, jnp.int32), 1, axis=1) & wm_l, _f32)
    ul0 = _shift_h(dc) + rs
    u0_ref[...] = u0.astype(_bf16)
    ul0_ref[...] = ul0.astype(_bf16)


def _main_body(u0_ref, ul0_ref, uw1_ref, ub1_ref, uw2_ref, ub2_ref,
               lw1_ref, lb1_ref, lw2_ref, lb2_ref,
               ulf_ref, u_s, ul_s):
    j = pl.program_id(1)
    wm_l, wm_r = _wmasks()

    @pl.when(j == 0)
    def _():
        u_s[...] = u0_ref[...].astype(_f32)
        ul_s[...] = ul0_ref[...].astype(_f32)

    u = u_s[...]
    ul = ul_s[...]

    xe = _celu(u)
    vm = _variants(xe, (-1, 0, 1), wm_l, wm_r, _CDT)
    c1 = _conv(uw1_ref[0], vm, (-1, 0, 1), True, ub1_ref[0],
               scale=_INV_WSCALE)
    xe2 = _celu(c1)
    vm2 = _variants(xe2, (-1, 0, 1), wm_l, wm_r)
    c2 = _conv(uw2_ref[0], vm2, (-1, 0, 1), True, ub2_ref[0])
    u_new = u + c2[:_F] * jax.nn.sigmoid(c2[_F:])
    u_s[...] = u_new

    ye = _celu(ul)
    vm3 = _variants(ye, (-1, 0), wm_l, wm_r)
    a = _celu(u_new).astype(_bf16)
    d1 = _conv(lw1_ref[0], vm3, (-1, 0), True, lb1_ref[0], extra=a)
    ye2 = _celu(d1)
    vm4 = _variants(ye2, (-1, 0), wm_l, wm_r)
    d2 = _conv(lw2_ref[0], vm4, (-1, 0), True, lb2_ref[0])
    ul_new = ul + d2[:_F] * jax.nn.sigmoid(d2[_F:])
    ul_s[...] = ul_new

    @pl.when(j == _NBLK - 1)
    def _():
        ulf_ref[...] = ul_new.astype(_bf16)


def _logistic_logp(xc, means, ls):
    inv = jnp.exp(-ls)
    centered = xc - means
    plus_in = inv * (centered + 1.0 / 255.0)
    min_in = inv * (centered - 1.0 / 255.0)
    cdf_delta = jax.nn.sigmoid(plus_in) - jax.nn.sigmoid(min_in)
    log_cdf_plus = plus_in - jax.nn.softplus(plus_in)
    log_om_cdf = -jax.nn.softplus(min_in)
    mid_in = inv * centered
    log_pdf_mid = mid_in - ls - 2.0 * jax.nn.softplus(mid_in)
    return jnp.where(
        xc < -0.999, log_cdf_plus,
        jnp.where(xc > 0.999, log_om_cdf,
                  jnp.where(cdf_delta > 1e-5,
                            jnp.log(jnp.maximum(cdf_delta, 1e-12)),
                            log_pdf_mid - np.float32(np.log(127.5)))))


def _epi_body(ulf_ref, x3_ref, ow_ref, ob_ref, out_ref):
    ul = ulf_ref[...].astype(_f32)
    z = _elu(ul).astype(_bf16)
    l = jnp.dot(ow_ref[...], z, preferred_element_type=_f32) + ob_ref[...]
    x = x3_ref[...]
    x0, x1 = x[0:1], x[1:2]
    xs = (x[0:1], x[1:2], x[2:3])
    m1 = l[10:20]
    m2 = l[40:50] + jnp.tanh(l[30:40]) * x0
    m3 = l[70:80] + jnp.tanh(l[60:70]) * x0 + jnp.tanh(l[90:100]) * x1
    lss = (jnp.maximum(l[20:30], -7.0),
           jnp.maximum(l[50:60], -7.0),
           jnp.maximum(l[80:90], -7.0))
    acc = (_logistic_logp(xs[0], m1, lss[0])
           + _logistic_logp(xs[1], m2, lss[1])
           + _logistic_logp(xs[2], m3, lss[2]))
    lg = l[0:10]
    lgm = lg - jnp.max(lg, axis=0, keepdims=True)
    lsm = lgm - jnp.log(jnp.sum(jnp.exp(lgm), axis=0, keepdims=True))
    tot = acc + lsm
    tm = jnp.max(tot, axis=0, keepdims=True)
    lse = tm + jnp.log(jnp.sum(jnp.exp(tot - tm), axis=0, keepdims=True))
    s = jnp.sum(lse, axis=1, keepdims=True)
    out_ref[...] = jnp.broadcast_to(s[None], (1, 1, 128))


def _pix(a):
    c = a.shape[1]
    return (a.reshape(_NB, _BB, c, _H, _W)
             .transpose(2, 0, 3, 1, 4)
             .reshape(c, _NT))


def _impl(samples, w_u_init, b_u_init, w_ul_d, b_ul_d, w_ul_dr, b_ul_dr,
          u_c1_w, u_c1_b, u_c2_w, u_c2_b,
          ul_c1_w, ul_c1_b, ul_nin_w, ul_nin_b, ul_c2_w, ul_c2_b,
          nin_out_w, nin_out_b, interpret=False):
    x = samples * 2.0 - 1.0
    xp = jnp.concatenate([x, jnp.ones_like(x[:, :1])], axis=1)
    xq = _pix(xp)
    x3 = _pix(x)

    def kstack(w):
        o = w.shape[0]
        return w.transpose(0, 3, 2, 1).reshape(o, -1).astype(_bf16)

    def kstack5(w):
        n, o = w.shape[0], w.shape[1]
        return (w.transpose(0, 1, 4, 3, 2).reshape(n, o, -1)
                * _WSCALE).astype(_CDT)

    wu0 = kstack(w_u_init)
    wd0 = kstack(w_ul_d)
    wr0 = kstack(w_ul_dr)
    bu0 = b_u_init.reshape(_F, 1)
    bd0 = b_ul_d.reshape(_F, 1)
    br0 = b_ul_dr.reshape(_F, 1)
    uw1 = kstack5(u_c1_w)
    uw2 = (u_c2_w.transpose(0, 1, 4, 3, 2)
           .reshape(_NBLK, _F2, -1).astype(_bf16))
    lw1 = jnp.concatenate(
        [ul_c1_w.transpose(0, 1, 4, 3, 2).reshape(_NBLK, _F, -1).astype(_bf16),
         ul_nin_w.astype(_bf16)], axis=2)
    lw2 = (ul_c2_w.transpose(0, 1, 4, 3, 2)
           .reshape(_NBLK, _F2, -1).astype(_bf16))
    ub1 = u_c1_b.reshape(_NBLK, _F, 1)
    ub2 = u_c2_b.reshape(_NBLK, _F2, 1)
    lb1 = (ul_c1_b + ul_nin_b).reshape(_NBLK, _F, 1)
    lb2 = ul_c2_b.reshape(_NBLK, _F2, 1)
    ow = nin_out_w.astype(_bf16)
    ob = nin_out_b.reshape(_P, 1)

    cp1 = pltpu.CompilerParams(dimension_semantics=("parallel",),
                               vmem_limit_bytes=50 * 1024 * 1024)
    cp2 = pltpu.CompilerParams(dimension_semantics=("parallel", "arbitrary"),
                               vmem_limit_bytes=56 * 1024 * 1024)

    u0, ul0 = pl.pallas_call(
        _init_body,
        grid=(_NB,),
        in_specs=[
            pl.BlockSpec((4, _N), lambda i: (0, i)),
            pl.BlockSpec((_F, 24), lambda i: (0, 0)),
            pl.BlockSpec((_F, 12), lambda i: (0, 0)),
            pl.BlockSpec((_F, 8), lambda i: (0, 0)),
            pl.BlockSpec((_F, 1), lambda i: (0, 0)),
            pl.BlockSpec((_F, 1), lambda i: (0, 0)),
            pl.BlockSpec((_F, 1), lambda i: (0, 0)),
        ],
        out_specs=[pl.BlockSpec((_F, _N), lambda i: (0, i))] * 2,
        out_shape=[jax.ShapeDtypeStruct((_F, _NT), _bf16)] * 2,
        compiler_params=cp1,
        name="pixcnn_init",
        interpret=interpret,
    )(xq, wu0, wd0, wr0, bu0, bd0, br0)

    ulf = pl.pallas_call(
        _main_body,
        grid=(_NB, _NBLK),
        in_specs=[
            pl.BlockSpec((_F, _N), lambda i, j: (0, i)),
            pl.BlockSpec((_F, _N), lambda i, j: (0, i)),
            pl.BlockSpec((1, _F, 6 * _F2), lambda i, j: (j, 0, 0)),
            pl.BlockSpec((1, _F, 1), lambda i, j: (j, 0, 0)),
            pl.BlockSpec((1, _F2, 6 * _F2), lambda i, j: (j, 0, 0)),
            pl.BlockSpec((1, _F2, 1), lambda i, j: (j, 0, 0)),
            pl.BlockSpec((1, _F, 5 * _F2), lambda i, j: (j, 0, 0)),
            pl.BlockSpec((1, _F, 1), lambda i, j: (j, 0, 0)),
            pl.BlockSpec((1, _F2, 4 * _F2), lambda i, j: (j, 0, 0)),
            pl.BlockSpec((1, _F2, 1), lambda i, j: (j, 0, 0)),
        ],
        out_specs=pl.BlockSpec((_F, _N), lambda i, j: (0, i)),
        out_shape=jax.ShapeDtypeStruct((_F, _NT), _bf16),
        scratch_shapes=[pltpu.VMEM((_F, _N), _f32),
                        pltpu.VMEM((_F, _N), _f32)],
        compiler_params=cp2,
        name="pixcnn_resnets",
        interpret=interpret,
    )(u0, ul0, uw1, ub1, uw2, ub2, lw1, lb1, lw2, lb2)

    parts = pl.pallas_call(
        _epi_body,
        grid=(_NB,),
        in_specs=[
            pl.BlockSpec((_F, _N), lambda i: (0, i)),
            pl.BlockSpec((3, _N), lambda i: (0, i)),
            pl.BlockSpec((_P, _F), lambda i: (0, 0)),
            pl.BlockSpec((_P, 1), lambda i: (0, 0)),
        ],
        out_specs=pl.BlockSpec((1, 1, 128), lambda i: (i, 0, 0)),
        out_shape=jax.ShapeDtypeStruct((_NB, 1, 128), _f32),
        compiler_params=cp1,
        name="pixcnn_dmll",
        interpret=interpret,
    )(ulf, x3, ow, ob)

    return jnp.sum(parts[:, 0, 0])


def kernel(samples, w_u_init, b_u_init, w_ul_d, b_ul_d, w_ul_dr, b_ul_dr,
           u_c1_w, u_c1_b, u_c2_w, u_c2_b,
           ul_c1_w, ul_c1_b, ul_nin_w, ul_nin_b, ul_c2_w, ul_c2_b,
           nin_out_w, nin_out_b):
    return _impl(samples, w_u_init, b_u_init, w_ul_d, b_ul_d, w_ul_dr,
                 b_ul_dr, u_c1_w, u_c1_b, u_c2_w, u_c2_b,
                 ul_c1_w, ul_c1_b, ul_nin_w, ul_nin_b, ul_c2_w, ul_c2_b,
                 nin_out_w, nin_out_b)
